# baseline (speedup 1.0000x reference)
import functools

import jax
import jax.numpy as jnp
from jax import lax
from jax.experimental import pallas as pl
from jax.experimental.pallas import tpu as pltpu
from jax.experimental.pallas import tpu_sc as plsc

D = 128
H = 150
CTX = 200
L = 16
RPT = 16
NT = -(-CTX // RPT)
NFULL = CTX // RPT
TAIL = CTX - NFULL * RPT
ND = D // L


def _sc_body(idx_hbm, table_hbm, out_hbm,
             idxa_v, idxt_v, rowsa_v, part_v, sem_i, sem_g):
    s = lax.axis_index("s")

    @pl.when(s < NFULL)
    def _full():
        pltpu.make_async_copy(idx_hbm.at[pl.ds(s * RPT, RPT)], idxa_v,
                              sem_i).start()
        pltpu.make_async_copy(idx_hbm.at[pl.ds(s * RPT, RPT)], idxa_v,
                              sem_i).wait()
        pltpu.make_async_copy(table_hbm.at[idxa_v], rowsa_v, sem_g).start()
        pltpu.make_async_copy(table_hbm.at[idxa_v], rowsa_v, sem_g).wait()
        for k in range(ND):
            acc = rowsa_v[0, pl.ds(k * L, L)]
            for r in range(1, RPT):
                acc = acc + rowsa_v[r, pl.ds(k * L, L)]
            part_v[pl.ds(k * L, L)] = acc
        pltpu.sync_copy(part_v, out_hbm.at[s])

    @pl.when(s == NFULL)
    def _tail():
        pltpu.make_async_copy(idx_hbm.at[pl.ds(NFULL * RPT, TAIL)], idxt_v,
                              sem_i).start()
        pltpu.make_async_copy(idx_hbm.at[pl.ds(NFULL * RPT, TAIL)], idxt_v,
                              sem_i).wait()
        pltpu.make_async_copy(table_hbm.at[idxt_v],
                              rowsa_v.at[pl.ds(0, TAIL)], sem_g).start()
        pltpu.make_async_copy(table_hbm.at[idxt_v],
                              rowsa_v.at[pl.ds(0, TAIL)], sem_g).wait()
        for k in range(ND):
            acc = rowsa_v[0, pl.ds(k * L, L)]
            for r in range(1, TAIL):
                acc = acc + rowsa_v[r, pl.ds(k * L, L)]
            part_v[pl.ds(k * L, L)] = acc
        pltpu.sync_copy(part_v, out_hbm.at[NFULL])


@functools.cache
def _sc_pool():
    return pl.kernel(
        _sc_body,
        mesh=plsc.VectorSubcoreMesh(core_axis_name="c", subcore_axis_name="s",
                                    num_cores=1),
        compiler_params=pltpu.CompilerParams(use_tc_tiling_on_sc=False),
        out_type=jax.ShapeDtypeStruct((NT, D), jnp.float32),
        scratch_types=[
            pltpu.VMEM((RPT,), jnp.int32),
            pltpu.VMEM((TAIL,), jnp.int32),
            pltpu.VMEM((RPT, D), jnp.float32),
            pltpu.VMEM((D,), jnp.float32),
            pltpu.SemaphoreType.DMA,
            pltpu.SemaphoreType.DMA,
        ],
    )


def _mlp_body(p_ref, w1_ref, b1_ref, w2_ref, b2_ref, out_ref):
    pooled = jnp.sum(p_ref[...], axis=0, keepdims=True)
    h = jnp.dot(pooled, w1_ref[...], preferred_element_type=jnp.float32)
    h = jnp.maximum(h + b1_ref[...], 0.0)
    logits = jnp.dot(h, w2_ref[...], preferred_element_type=jnp.float32)
    logits = logits + b2_ref[...]
    m = jnp.max(logits, axis=-1, keepdims=True)
    x = logits - m
    lse = jnp.log(jnp.sum(jnp.exp(x), axis=-1, keepdims=True))
    out_ref[...] = x - lse


_mlp = pl.pallas_call(
    _mlp_body,
    out_shape=jax.ShapeDtypeStruct((1, D), jnp.float32),
)


def kernel(input, emb_table, W1, b1, W2, b2):
    idx = input.astype(jnp.int32)
    parts = _sc_pool()(idx, emb_table)
    return _mlp(parts, W1, b1.reshape(1, H), W2, b2.reshape(1, D))
